# Initial kernel scaffold; baseline (speedup 1.0000x reference)
import jax
import jax.numpy as jnp
from jax.experimental import pallas as pl


def kernel(x, edge_index, W0, b0, W1, b1, W2, b2):
    raise NotImplementedError("write your pallas kernel here")


# speedup vs baseline: 17.4092x; 17.4092x over previous
import functools

import jax
import jax.numpy as jnp
from jax import lax
from jax.experimental import pallas as pl
from jax.experimental.pallas import tpu as pltpu
from jax.experimental.pallas import tpu_sc as plsc

NN = 10000
NPAD = 10240
NE = 320000
DH = 16
DIN = 128

NC, NS = 2, 16
NW = NC * NS
CHUNK = 128
EPT = 10112
NCHUNK = EPT // CHUNK
EPAD = EPT * NW
RPT = NPAD // NS

_mesh = plsc.VectorSubcoreMesh(core_axis_name="c", subcore_axis_name="s")


def _sc_pass_body(with_gather, g_hbm, src_hbm, dst_hbm, out_hbm,
                  idx_v, dst_v, rows_v, acc_sh, sem):
    cid = lax.axis_index("c")
    sid = lax.axis_index("s")
    wid = sid * NC + cid

    def _zrow(i, _):
        rows_v[i] = jnp.zeros((16,), jnp.float32)
        return 0
    lax.fori_loop(0, CHUNK, _zrow, 0)
    for r in range(RPT // CHUNK):
        pltpu.sync_copy(rows_v, acc_sh.at[pl.ds(sid * RPT + r * CHUNK, CHUNK)])
    if not with_gather:
        def _orow(i, _):
            rows_v[i] = jnp.ones((16,), jnp.float32)
            return 0
        lax.fori_loop(0, CHUNK, _orow, 0)
    plsc.subcore_barrier()

    base = wid * EPT

    def _echunk(i, _):
        off = base + i * CHUNK
        pltpu.sync_copy(dst_hbm.at[pl.ds(off, CHUNK)], dst_v)
        if with_gather:
            pltpu.sync_copy(src_hbm.at[pl.ds(off, CHUNK)], idx_v)
            pltpu.async_copy(g_hbm.at[idx_v], rows_v, sem).wait()
        pltpu.sync_copy(rows_v, acc_sh.at[dst_v], add=True)
        return 0
    lax.fori_loop(0, NCHUNK, _echunk, 0)
    plsc.subcore_barrier()

    for r in range(RPT // CHUNK):
        row = sid * RPT + r * CHUNK
        pltpu.sync_copy(acc_sh.at[pl.ds(row, CHUNK)],
                        out_hbm.at[cid, pl.ds(row, CHUNK)])


def _make_sc_pass(with_gather):
    return pl.kernel(
        functools.partial(_sc_pass_body, with_gather),
        out_type=jax.ShapeDtypeStruct((NC, NPAD, DH), jnp.float32),
        mesh=_mesh,
        scratch_types=[
            pltpu.VMEM((CHUNK,), jnp.int32),
            pltpu.VMEM((CHUNK,), jnp.int32),
            pltpu.VMEM((CHUNK, DH), jnp.float32),
            pltpu.VMEM_SHARED((NPAD, DH), jnp.float32),
            pltpu.SemaphoreType.DMA,
        ],
        compiler_params=pltpu.CompilerParams(use_tc_tiling_on_sc=False),
    )


_sc_edge_pass = _make_sc_pass(True)
_sc_deg_pass = _make_sc_pass(False)


def _tc_prep_body(x_ref, w_ref, d0_ref, d1_ref, hw_ref, g_ref, dinv_ref):
    deg = d0_ref[...] + d1_ref[...] + 1.0
    dinv = lax.rsqrt(deg)
    hw = jnp.dot(x_ref[...], w_ref[...], preferred_element_type=jnp.float32)
    hw_ref[...] = hw
    g_ref[...] = dinv * hw
    dinv_ref[...] = dinv


def _tc_layer_body(last, a0_ref, a1_ref, hwp_ref, dinv_ref, b_ref, w_ref, *outs):
    dinv = dinv_ref[...]
    h = dinv * (a0_ref[...] + a1_ref[...]) + dinv * dinv * hwp_ref[...] + b_ref[...]
    if last:
        outs[0][...] = h
    else:
        h = jnp.maximum(h, 0.0)
        hw = jnp.dot(h, w_ref[...], preferred_element_type=jnp.float32)
        outs[0][...] = hw
        outs[1][...] = dinv * hw


def kernel(x, edge_index, W0, b0, W1, b1, W2, b2):
    src = edge_index[0].astype(jnp.int32)
    dst = edge_index[1].astype(jnp.int32)
    pad = EPAD - NE
    src = jnp.concatenate([src, jnp.zeros((pad,), jnp.int32)])
    dst = jnp.concatenate([dst, jnp.full((pad,), NN, jnp.int32)])
    x_pad = jnp.pad(x, ((0, NPAD - NN), (0, 0)))
    g_dummy = jnp.zeros((NPAD, DH), jnp.float32)

    f32 = jnp.float32
    nd16 = jax.ShapeDtypeStruct((NPAD, DH), f32)

    degp = _sc_deg_pass(g_dummy, src, dst)

    hw, g, dinv = pl.pallas_call(
        _tc_prep_body,
        out_shape=[nd16, nd16, nd16],
    )(x_pad, W0, degp[0], degp[1])

    for (W_next, b_prev, last) in ((W1, b0, False), (W2, b1, False), (None, b2, True)):
        accp = _sc_edge_pass(g, src, dst)
        b2d = b_prev.reshape(1, DH)
        if last:
            out = pl.pallas_call(
                functools.partial(_tc_layer_body, True),
                out_shape=[nd16],
            )(accp[0], accp[1], hw, dinv, b2d, W2)[0]
        else:
            hw, g = pl.pallas_call(
                functools.partial(_tc_layer_body, False),
                out_shape=[nd16, nd16],
            )(accp[0], accp[1], hw, dinv, b2d, W_next)

    return out[:NN]


# speedup vs baseline: 27.9517x; 1.6056x over previous
import functools

import jax
import jax.numpy as jnp
from jax import lax
from jax.experimental import pallas as pl
from jax.experimental.pallas import tpu as pltpu
from jax.experimental.pallas import tpu_sc as plsc

NN = 10000
NPAD = 10240
NE = 320000
DH = 16
DIN = 128

NC, NS = 2, 16
NW = NC * NS
CHUNK = 128
NCHUNK = 80
EPT = NCHUNK * CHUNK
EPAD = EPT * NW
RPT = NPAD // NS
KG = 10
NGRP = NCHUNK // KG

_mesh = plsc.VectorSubcoreMesh(core_axis_name="c", subcore_axis_name="s")


def _sc_pass_body(with_gather, g_hbm, src_hbm, dst_hbm, out_hbm,
                  src_v, dst_v, rows, acc_sh, sem_g, sem_s):
    cid = lax.axis_index("c")
    sid = lax.axis_index("s")
    wid = sid * NC + cid

    def _zrow(i, _):
        rows[0, i] = jnp.zeros((DH,), jnp.float32)
        return 0
    lax.fori_loop(0, CHUNK, _zrow, 0)
    for r in range(RPT // CHUNK):
        pltpu.sync_copy(rows.at[0], acc_sh.at[pl.ds(sid * RPT + r * CHUNK, CHUNK)])

    pltpu.sync_copy(dst_hbm.at[wid], dst_v)
    if with_gather:
        pltpu.sync_copy(src_hbm.at[wid], src_v)
    else:
        def _orow(i, _):
            rows[0, i] = jnp.ones((DH,), jnp.float32)
            return 0
        lax.fori_loop(0, CHUNK, _orow, 0)
    plsc.subcore_barrier()

    if with_gather:
        def _grp(g, _):
            c0 = g * KG
            gd = [pltpu.async_copy(g_hbm.at[src_v.at[c0 + j]], rows.at[j], sem_g)
                  for j in range(KG)]
            for d in gd:
                d.wait()
            sd = [pltpu.async_copy(rows.at[j], acc_sh.at[dst_v.at[c0 + j]],
                                   sem_s, add=True)
                  for j in range(KG)]
            for d in sd:
                d.wait()
            return 0
    else:
        def _grp(g, _):
            c0 = g * KG
            sd = [pltpu.async_copy(rows.at[0], acc_sh.at[dst_v.at[c0 + j]],
                                   sem_s, add=True)
                  for j in range(KG)]
            for d in sd:
                d.wait()
            return 0
    lax.fori_loop(0, NGRP, _grp, 0)
    plsc.subcore_barrier()

    for r in range(RPT // CHUNK):
        row = sid * RPT + r * CHUNK
        pltpu.sync_copy(acc_sh.at[pl.ds(row, CHUNK)],
                        out_hbm.at[cid, pl.ds(row, CHUNK)])


def _make_sc_pass(with_gather):
    return pl.kernel(
        functools.partial(_sc_pass_body, with_gather),
        out_type=jax.ShapeDtypeStruct((NC, NPAD, DH), jnp.float32),
        mesh=_mesh,
        scratch_types=[
            pltpu.VMEM((NCHUNK, CHUNK), jnp.int32),
            pltpu.VMEM((NCHUNK, CHUNK), jnp.int32),
            pltpu.VMEM((KG, CHUNK, DH), jnp.float32),
            pltpu.VMEM_SHARED((NPAD, DH), jnp.float32),
            pltpu.SemaphoreType.DMA,
            pltpu.SemaphoreType.DMA,
        ],
        compiler_params=pltpu.CompilerParams(use_tc_tiling_on_sc=False),
    )


_sc_edge_pass = _make_sc_pass(True)
_sc_deg_pass = _make_sc_pass(False)


def _tc_prep_body(x_ref, w_ref, d0_ref, d1_ref, hw_ref, g_ref, dinv_ref):
    deg = d0_ref[...] + d1_ref[...] + 1.0
    dinv = lax.rsqrt(deg)
    hw = jnp.dot(x_ref[...], w_ref[...], preferred_element_type=jnp.float32)
    hw_ref[...] = hw
    g_ref[...] = dinv * hw
    dinv_ref[...] = dinv


def _tc_layer_body(last, a0_ref, a1_ref, hwp_ref, dinv_ref, b_ref, w_ref, *outs):
    dinv = dinv_ref[...]
    h = dinv * (a0_ref[...] + a1_ref[...]) + dinv * dinv * hwp_ref[...] + b_ref[...]
    if last:
        outs[0][...] = h
    else:
        h = jnp.maximum(h, 0.0)
        hw = jnp.dot(h, w_ref[...], preferred_element_type=jnp.float32)
        outs[0][...] = hw
        outs[1][...] = dinv * hw


def kernel(x, edge_index, W0, b0, W1, b1, W2, b2):
    src = edge_index[0].astype(jnp.int32)
    dst = edge_index[1].astype(jnp.int32)
    pad = EPAD - NE
    src = jnp.concatenate([src, jnp.zeros((pad,), jnp.int32)])
    dst = jnp.concatenate([dst, jnp.full((pad,), NN, jnp.int32)])
    src = src.reshape(NW, NCHUNK, CHUNK)
    dst = dst.reshape(NW, NCHUNK, CHUNK)
    x_pad = jnp.pad(x, ((0, NPAD - NN), (0, 0)))
    g_dummy = jnp.zeros((NPAD, DH), jnp.float32)

    f32 = jnp.float32
    nd16 = jax.ShapeDtypeStruct((NPAD, DH), f32)

    degp = _sc_deg_pass(g_dummy, src, dst)

    hw, g, dinv = pl.pallas_call(
        _tc_prep_body,
        out_shape=[nd16, nd16, nd16],
    )(x_pad, W0, degp[0], degp[1])

    for (W_next, b_prev, last) in ((W1, b0, False), (W2, b1, False), (None, b2, True)):
        accp = _sc_edge_pass(g, src, dst)
        b2d = b_prev.reshape(1, DH)
        if last:
            out = pl.pallas_call(
                functools.partial(_tc_layer_body, True),
                out_shape=[nd16],
            )(accp[0], accp[1], hw, dinv, b2d, W2)[0]
        else:
            hw, g = pl.pallas_call(
                functools.partial(_tc_layer_body, False),
                out_shape=[nd16, nd16],
            )(accp[0], accp[1], hw, dinv, b2d, W_next)

    return out[:NN]


# speedup vs baseline: 41.3037x; 1.4777x over previous
import functools

import jax
import jax.numpy as jnp
from jax import lax
from jax.experimental import pallas as pl
from jax.experimental.pallas import tpu as pltpu
from jax.experimental.pallas import tpu_sc as plsc

NN = 10000
NPAD = 10240
NE = 320000
DH = 16
DIN = 128

NC, NS = 2, 16
NW = NC * NS
CHUNK = 128
NCHUNK = 80
EPT = NCHUNK * CHUNK
EPAD = EPT * NW
RPT = NPAD // NS
KG = 10
NGRP = NCHUNK // KG

_mesh = plsc.VectorSubcoreMesh(core_axis_name="c", subcore_axis_name="s")


def _sc_pass_body(with_gather, g_hbm, src_hbm, dst_hbm, out_hbm,
                  src_v, dst_v, rows, acc_sh, g_sh, sem_g, sem_s):
    cid = lax.axis_index("c")
    sid = lax.axis_index("s")
    wid = sid * NC + cid

    def _zrow(i, _):
        rows[0, i] = jnp.zeros((DH,), jnp.float32)
        return 0
    lax.fori_loop(0, CHUNK, _zrow, 0)
    for r in range(RPT // CHUNK):
        pltpu.sync_copy(rows.at[0], acc_sh.at[pl.ds(sid * RPT + r * CHUNK, CHUNK)])

    pltpu.sync_copy(dst_hbm.at[wid], dst_v)
    if with_gather:
        pltpu.sync_copy(src_hbm.at[wid], src_v)
        pltpu.sync_copy(g_hbm.at[pl.ds(sid * RPT, RPT)],
                        g_sh.at[pl.ds(sid * RPT, RPT)])
    else:
        def _orow(i, _):
            rows[0, i] = jnp.ones((DH,), jnp.float32)
            return 0
        lax.fori_loop(0, CHUNK, _orow, 0)
    plsc.subcore_barrier()

    if with_gather:
        def _grp(g, _):
            c0 = g * KG
            gd = [pltpu.async_copy(g_sh.at[src_v.at[c0 + j]], rows.at[j], sem_g)
                  for j in range(KG)]
            for d in gd:
                d.wait()
            sd = [pltpu.async_copy(rows.at[j], acc_sh.at[dst_v.at[c0 + j]],
                                   sem_s, add=True)
                  for j in range(KG)]
            for d in sd:
                d.wait()
            return 0
    else:
        def _grp(g, _):
            c0 = g * KG
            sd = [pltpu.async_copy(rows.at[0], acc_sh.at[dst_v.at[c0 + j]],
                                   sem_s, add=True)
                  for j in range(KG)]
            for d in sd:
                d.wait()
            return 0
    lax.fori_loop(0, NGRP, _grp, 0)
    plsc.subcore_barrier()

    for r in range(RPT // CHUNK):
        row = sid * RPT + r * CHUNK
        pltpu.sync_copy(acc_sh.at[pl.ds(row, CHUNK)],
                        out_hbm.at[cid, pl.ds(row, CHUNK)])


def _make_sc_pass(with_gather):
    return pl.kernel(
        functools.partial(_sc_pass_body, with_gather),
        out_type=jax.ShapeDtypeStruct((NC, NPAD, DH), jnp.float32),
        mesh=_mesh,
        scratch_types=[
            pltpu.VMEM((NCHUNK, CHUNK), jnp.int32),
            pltpu.VMEM((NCHUNK, CHUNK), jnp.int32),
            pltpu.VMEM((KG, CHUNK, DH), jnp.float32),
            pltpu.VMEM_SHARED((NPAD, DH), jnp.float32),
            pltpu.VMEM_SHARED((NPAD, DH), jnp.float32),
            pltpu.SemaphoreType.DMA,
            pltpu.SemaphoreType.DMA,
        ],
        compiler_params=pltpu.CompilerParams(use_tc_tiling_on_sc=False),
    )


_sc_edge_pass = _make_sc_pass(True)
_sc_deg_pass = _make_sc_pass(False)


def _tc_prep_body(x_ref, w_ref, d0_ref, d1_ref, hw_ref, g_ref, dinv_ref):
    deg = d0_ref[...] + d1_ref[...] + 1.0
    dinv = lax.rsqrt(deg)
    hw = jnp.dot(x_ref[...], w_ref[...], preferred_element_type=jnp.float32)
    hw_ref[...] = hw
    g_ref[...] = dinv * hw
    dinv_ref[...] = dinv


def _tc_layer_body(last, a0_ref, a1_ref, hwp_ref, dinv_ref, b_ref, w_ref, *outs):
    dinv = dinv_ref[...]
    h = dinv * (a0_ref[...] + a1_ref[...]) + dinv * dinv * hwp_ref[...] + b_ref[...]
    if last:
        outs[0][...] = h
    else:
        h = jnp.maximum(h, 0.0)
        hw = jnp.dot(h, w_ref[...], preferred_element_type=jnp.float32)
        outs[0][...] = hw
        outs[1][...] = dinv * hw


def kernel(x, edge_index, W0, b0, W1, b1, W2, b2):
    src = edge_index[0].astype(jnp.int32)
    dst = edge_index[1].astype(jnp.int32)
    pad = EPAD - NE
    src = jnp.concatenate([src, jnp.zeros((pad,), jnp.int32)])
    dst = jnp.concatenate([dst, jnp.full((pad,), NN, jnp.int32)])
    src = src.reshape(NW, NCHUNK, CHUNK)
    dst = dst.reshape(NW, NCHUNK, CHUNK)
    x_pad = jnp.pad(x, ((0, NPAD - NN), (0, 0)))
    g_dummy = jnp.zeros((NPAD, DH), jnp.float32)

    f32 = jnp.float32
    nd16 = jax.ShapeDtypeStruct((NPAD, DH), f32)

    degp = _sc_deg_pass(g_dummy, src, dst)

    hw, g, dinv = pl.pallas_call(
        _tc_prep_body,
        out_shape=[nd16, nd16, nd16],
    )(x_pad, W0, degp[0], degp[1])

    for (W_next, b_prev, last) in ((W1, b0, False), (W2, b1, False), (None, b2, True)):
        accp = _sc_edge_pass(g, src, dst)
        b2d = b_prev.reshape(1, DH)
        if last:
            out = pl.pallas_call(
                functools.partial(_tc_layer_body, True),
                out_shape=[nd16],
            )(accp[0], accp[1], hw, dinv, b2d, W2)[0]
        else:
            hw, g = pl.pallas_call(
                functools.partial(_tc_layer_body, False),
                out_shape=[nd16, nd16],
            )(accp[0], accp[1], hw, dinv, b2d, W_next)

    return out[:NN]


# speedup vs baseline: 64.6690x; 1.5657x over previous
import functools

import jax
import jax.numpy as jnp
from jax import lax
from jax.experimental import pallas as pl
from jax.experimental.pallas import tpu as pltpu
from jax.experimental.pallas import tpu_sc as plsc

NN = 10000
NPAD = 10240
NE = 320000
DH = 16
DIN = 128

NC, NS = 2, 16
NW = NC * NS
CHUNK = 128
NCHUNK = 80
EPT = NCHUNK * CHUNK
EPAD = EPT * NW
RPT = NPAD // NS
KG = 10
NGRP = NCHUNK // KG

_mesh = plsc.VectorSubcoreMesh(core_axis_name="c", subcore_axis_name="s")


def _sc_pass_body(with_gather, g_hbm, src_hbm, dst_hbm, out_hbm,
                  src_v, dst_v, rows, acc_sh, g_sh, sem_g, sem_s):
    cid = lax.axis_index("c")
    sid = lax.axis_index("s")
    wid = sid * NC + cid

    def _zrow(i, _):
        rows[0, i] = jnp.zeros((DH,), jnp.float32)
        return 0
    lax.fori_loop(0, CHUNK, _zrow, 0)
    for r in range(RPT // CHUNK):
        pltpu.sync_copy(rows.at[0], acc_sh.at[pl.ds(sid * RPT + r * CHUNK, CHUNK)])

    pltpu.sync_copy(dst_hbm.at[wid], dst_v)
    if with_gather:
        pltpu.sync_copy(src_hbm.at[wid], src_v)
        pltpu.sync_copy(g_hbm.at[pl.ds(sid * RPT, RPT)],
                        g_sh.at[pl.ds(sid * RPT, RPT)])
    else:
        def _orow(i, _):
            rows[0, i] = jnp.ones((DH,), jnp.float32)
            return 0
        lax.fori_loop(0, CHUNK, _orow, 0)
    plsc.subcore_barrier()

    if with_gather:
        def _grp(g, _):
            c0 = g * KG
            gd = [pltpu.async_copy(g_sh.at[src_v.at[c0 + j]], rows.at[j], sem_g)
                  for j in range(KG)]
            for d in gd:
                d.wait()
            sd = [pltpu.async_copy(rows.at[j], acc_sh.at[dst_v.at[c0 + j]],
                                   sem_s, add=True)
                  for j in range(KG)]
            for d in sd:
                d.wait()
            return 0
    else:
        def _grp(g, _):
            c0 = g * KG
            sd = [pltpu.async_copy(rows.at[0], acc_sh.at[dst_v.at[c0 + j]],
                                   sem_s, add=True)
                  for j in range(KG)]
            for d in sd:
                d.wait()
            return 0
    lax.fori_loop(0, NGRP, _grp, 0)
    plsc.subcore_barrier()

    for r in range(RPT // CHUNK):
        row = sid * RPT + r * CHUNK
        pltpu.sync_copy(acc_sh.at[pl.ds(row, CHUNK)],
                        out_hbm.at[cid, pl.ds(row, CHUNK)])


def _make_sc_pass(with_gather):
    return pl.kernel(
        functools.partial(_sc_pass_body, with_gather),
        out_type=jax.ShapeDtypeStruct((NC, NPAD, DH), jnp.float32),
        mesh=_mesh,
        scratch_types=[
            pltpu.VMEM((NCHUNK, CHUNK), jnp.int32),
            pltpu.VMEM((NCHUNK, CHUNK), jnp.int32),
            pltpu.VMEM((KG, CHUNK, DH), jnp.float32),
            pltpu.VMEM_SHARED((NPAD, DH), jnp.float32),
            pltpu.VMEM_SHARED((NPAD, DH), jnp.float32),
            pltpu.SemaphoreType.DMA,
            pltpu.SemaphoreType.DMA,
        ],
        compiler_params=pltpu.CompilerParams(use_tc_tiling_on_sc=False),
    )


_sc_edge_pass = _make_sc_pass(True)
_sc_deg_pass = _make_sc_pass(False)


NP8 = NPAD // 8
NR8 = NN // 8


def _tc_prep_body(x8_ref, w0b_ref, degp_ref, hw_ref, g_ref, dinv_ref):
    deg = degp_ref[0] + degp_ref[1] + 1.0
    dinv = lax.rsqrt(deg)
    dinv_ref[...] = dinv
    hw = jnp.dot(x8_ref[...], w0b_ref[...], preferred_element_type=jnp.float32)
    hw = jnp.concatenate([hw, jnp.zeros((NP8 - NR8, 128), jnp.float32)])
    hw_ref[...] = hw
    g_ref[...] = dinv * hw


def _tc_layer_body(accp_ref, hwp_ref, dinv_ref, b_ref, w_ref, hw_ref, g_ref):
    dinv = dinv_ref[...]
    h = dinv * (accp_ref[0] + accp_ref[1]) + dinv * dinv * hwp_ref[...] + b_ref[...]
    h = jnp.maximum(h, 0.0)
    hw = jnp.dot(h, w_ref[...], preferred_element_type=jnp.float32)
    hw_ref[...] = hw
    g_ref[...] = dinv * hw


def _tc_final_body(accp_ref, hwp_ref, dinv_ref, b_ref, out_ref):
    dinv = dinv_ref[...]
    out_ref[...] = (dinv * (accp_ref[0] + accp_ref[1])
                    + dinv * dinv * hwp_ref[...] + b_ref[...])


def kernel(x, edge_index, W0, b0, W1, b1, W2, b2):
    src = edge_index[0].astype(jnp.int32)
    dst = edge_index[1].astype(jnp.int32)
    pad = EPAD - NE
    src = jnp.concatenate([src, jnp.zeros((pad,), jnp.int32)])
    dst = jnp.concatenate([dst, jnp.full((pad,), NN, jnp.int32)])
    src = src.reshape(NW, NCHUNK, CHUNK)
    dst = dst.reshape(NW, NCHUNK, CHUNK)

    f32 = jnp.float32
    x8 = x.reshape(NR8, 8 * DIN)
    eye8 = jnp.eye(8, dtype=f32)
    W0b = jnp.kron(eye8, W0)
    Wb = [jnp.kron(eye8, W1), jnp.kron(eye8, W2), None]
    bb = [jnp.tile(b0, 8).reshape(1, 128), jnp.tile(b1, 8).reshape(1, 128),
          jnp.tile(b2, 8).reshape(1, 128)]
    g_dummy = jnp.zeros((NPAD, DH), f32)

    phys = jax.ShapeDtypeStruct((NP8, 128), f32)

    degp = _sc_deg_pass(g_dummy, src, dst).reshape(NC, NP8, 128)

    hw, g, dinv = pl.pallas_call(
        _tc_prep_body,
        out_shape=[phys, phys, phys],
    )(x8, W0b, degp)

    for i, last in ((0, False), (1, False), (2, True)):
        accp = _sc_edge_pass(g.reshape(NPAD, DH), src, dst).reshape(NC, NP8, 128)
        if last:
            out = pl.pallas_call(
                _tc_final_body,
                out_shape=[phys],
            )(accp, hw, dinv, bb[i])[0]
        else:
            hw, g = pl.pallas_call(
                _tc_layer_body,
                out_shape=[phys, phys],
            )(accp, hw, dinv, bb[i], Wb[i])

    return out.reshape(NPAD, DH)[:NN]


# speedup vs baseline: 75.9724x; 1.1748x over previous
import functools

import jax
import jax.numpy as jnp
from jax import lax
from jax.experimental import pallas as pl
from jax.experimental.pallas import tpu as pltpu
from jax.experimental.pallas import tpu_sc as plsc

NN = 10000
NPAD = 10240
NE = 320000
DH = 16
DIN = 128

NC, NS = 2, 16
NW = NC * NS
CHUNK = 125
NCHUNK = 80
EPT = NCHUNK * CHUNK
ZCH = 128
RPT = NPAD // NS
KG = 10
NGRP = NCHUNK // KG

_mesh = plsc.VectorSubcoreMesh(core_axis_name="c", subcore_axis_name="s")


def _sc_pass_body(with_gather, g_hbm, src_hbm, dst_hbm, out_hbm,
                  src_v, dst_v, rows, zbuf, acc_sh, g_sh, sem_g, sem_s):
    cid = lax.axis_index("c")
    sid = lax.axis_index("s")
    wid = sid * NC + cid

    def _zrow(i, _):
        zbuf[i] = jnp.zeros((DH,), jnp.float32)
        return 0
    lax.fori_loop(0, ZCH, _zrow, 0)
    for r in range(RPT // ZCH):
        pltpu.sync_copy(zbuf, acc_sh.at[pl.ds(sid * RPT + r * ZCH, ZCH)])

    pltpu.sync_copy(dst_hbm.at[wid], dst_v)
    if with_gather:
        pltpu.sync_copy(src_hbm.at[wid], src_v)
        pltpu.sync_copy(g_hbm.at[pl.ds(sid * RPT, RPT)],
                        g_sh.at[pl.ds(sid * RPT, RPT)])
    else:
        def _orow(i, _):
            rows[0, i] = jnp.ones((DH,), jnp.float32)
            return 0
        lax.fori_loop(0, CHUNK, _orow, 0)
    plsc.subcore_barrier()

    if with_gather:
        def _grp(g, _):
            c0 = g * KG
            gd = [pltpu.async_copy(g_sh.at[src_v.at[c0 + j]], rows.at[j], sem_g)
                  for j in range(KG)]
            for d in gd:
                d.wait()
            sd = [pltpu.async_copy(rows.at[j], acc_sh.at[dst_v.at[c0 + j]],
                                   sem_s, add=True)
                  for j in range(KG)]
            for d in sd:
                d.wait()
            return 0
    else:
        def _grp(g, _):
            c0 = g * KG
            sd = [pltpu.async_copy(rows.at[0], acc_sh.at[dst_v.at[c0 + j]],
                                   sem_s, add=True)
                  for j in range(KG)]
            for d in sd:
                d.wait()
            return 0
    lax.fori_loop(0, NGRP, _grp, 0)
    plsc.subcore_barrier()

    for r in range(RPT // ZCH):
        row = sid * RPT + r * ZCH
        pltpu.sync_copy(acc_sh.at[pl.ds(row, ZCH)],
                        out_hbm.at[cid, pl.ds(row, ZCH)])


def _make_sc_pass(with_gather):
    return pl.kernel(
        functools.partial(_sc_pass_body, with_gather),
        out_type=jax.ShapeDtypeStruct((NC, NPAD, DH), jnp.float32),
        mesh=_mesh,
        scratch_types=[
            pltpu.VMEM((NCHUNK, CHUNK), jnp.int32),
            pltpu.VMEM((NCHUNK, CHUNK), jnp.int32),
            pltpu.VMEM((KG, CHUNK, DH), jnp.float32),
            pltpu.VMEM((ZCH, DH), jnp.float32),
            pltpu.VMEM_SHARED((NPAD, DH), jnp.float32),
            pltpu.VMEM_SHARED((NPAD, DH), jnp.float32),
            pltpu.SemaphoreType.DMA,
            pltpu.SemaphoreType.DMA,
        ],
        compiler_params=pltpu.CompilerParams(use_tc_tiling_on_sc=False),
    )


_sc_edge_pass = _make_sc_pass(True)
_sc_deg_pass = _make_sc_pass(False)


NP8 = NPAD // 8
NR8 = NN // 8


def _tc_mm0_body(x8_ref, w0b_ref, hw_ref):
    hw = jnp.dot(x8_ref[...], w0b_ref[...], preferred_element_type=jnp.float32)
    hw_ref[...] = jnp.concatenate([hw, jnp.zeros((NP8 - NR8, 128), jnp.float32)])


def _tc_scale_body(degp_ref, hw_ref, g_ref, dinv_ref):
    deg = degp_ref[0] + degp_ref[1] + 1.0
    dinv = lax.rsqrt(deg)
    dinv_ref[...] = dinv
    g_ref[...] = dinv * hw_ref[...]


def _tc_layer_body(accp_ref, hwp_ref, dinv_ref, b_ref, w_ref, hw_ref, g_ref):
    dinv = dinv_ref[...]
    h = dinv * (accp_ref[0] + accp_ref[1]) + dinv * dinv * hwp_ref[...] + b_ref[...]
    h = jnp.maximum(h, 0.0)
    hw = jnp.dot(h, w_ref[...], preferred_element_type=jnp.float32)
    hw_ref[...] = hw
    g_ref[...] = dinv * hw


def _tc_final_body(accp_ref, hwp_ref, dinv_ref, b_ref, out_ref):
    dinv = dinv_ref[...]
    out_ref[...] = (dinv * (accp_ref[0] + accp_ref[1])
                    + dinv * dinv * hwp_ref[...] + b_ref[...])


def kernel(x, edge_index, W0, b0, W1, b1, W2, b2):
    src = edge_index[0].astype(jnp.int32).reshape(NW, NCHUNK, CHUNK)
    dst = edge_index[1].astype(jnp.int32).reshape(NW, NCHUNK, CHUNK)

    f32 = jnp.float32
    x8 = x.reshape(NR8, 8 * DIN)
    eye8 = jnp.eye(8, dtype=f32)
    W0b = jnp.kron(eye8, W0)
    Wb = [jnp.kron(eye8, W1), jnp.kron(eye8, W2), None]
    bb = [jnp.tile(b0, 8).reshape(1, 128), jnp.tile(b1, 8).reshape(1, 128),
          jnp.tile(b2, 8).reshape(1, 128)]
    g_dummy = jnp.zeros((NPAD, DH), f32)

    phys = jax.ShapeDtypeStruct((NP8, 128), f32)

    degp = _sc_deg_pass(g_dummy, src, dst).reshape(NC, NP8, 128)

    hw = pl.pallas_call(_tc_mm0_body, out_shape=[phys])(x8, W0b)[0]
    g, dinv = pl.pallas_call(
        _tc_scale_body,
        out_shape=[phys, phys],
    )(degp, hw)

    for i, last in ((0, False), (1, False), (2, True)):
        accp = _sc_edge_pass(g.reshape(NPAD, DH), src, dst).reshape(NC, NP8, 128)
        if last:
            out = pl.pallas_call(
                _tc_final_body,
                out_shape=[phys],
            )(accp, hw, dinv, bb[i])[0]
        else:
            hw, g = pl.pallas_call(
                _tc_layer_body,
                out_shape=[phys, phys],
            )(accp, hw, dinv, bb[i], Wb[i])

    return out.reshape(NPAD, DH)[:NN]


# speedup vs baseline: 77.4883x; 1.0200x over previous
import functools

import jax
import jax.numpy as jnp
from jax import lax
from jax.experimental import pallas as pl
from jax.experimental.pallas import tpu as pltpu
from jax.experimental.pallas import tpu_sc as plsc

NN = 10000
NPAD = 10240
NE = 320000
DH = 16
DIN = 128

NC, NS = 2, 16
NW = NC * NS
CHUNK = 80
NCHUNK = 125
EPT = NCHUNK * CHUNK
ZCH = 128
RPT = NPAD // NS
KG = 5
NGRP = NCHUNK // KG

_mesh = plsc.VectorSubcoreMesh(core_axis_name="c", subcore_axis_name="s")


def _sc_pass_body(with_gather, g_hbm, ei_hbm, out_hbm,
                  src_v, dst_v, rows, zbuf, acc_sh, g_sh, sem_g, sem_s):
    cid = lax.axis_index("c")
    sid = lax.axis_index("s")
    wid = sid * NC + cid

    def _zrow(i, _):
        zbuf[i] = jnp.zeros((DH,), jnp.float32)
        return 0
    lax.fori_loop(0, ZCH, _zrow, 0)
    for r in range(RPT // ZCH):
        pltpu.sync_copy(zbuf, acc_sh.at[pl.ds(sid * RPT + r * ZCH, ZCH)])

    base = wid * EPT
    pltpu.sync_copy(ei_hbm.at[1, pl.ds(base, EPT)], dst_v)
    if with_gather:
        pltpu.sync_copy(ei_hbm.at[0, pl.ds(base, EPT)], src_v)
        pltpu.sync_copy(g_hbm.at[pl.ds(sid * RPT, RPT)],
                        g_sh.at[pl.ds(sid * RPT, RPT)])
    else:
        def _orow(i, _):
            rows[0, i] = jnp.ones((DH,), jnp.float32)
            return 0
        lax.fori_loop(0, CHUNK, _orow, 0)
    plsc.subcore_barrier()

    if with_gather:
        def _grp(g, _):
            c0 = g * KG * CHUNK
            gd = [pltpu.async_copy(
                      g_sh.at[src_v.at[pl.ds(c0 + j * CHUNK, CHUNK)]],
                      rows.at[j], sem_g)
                  for j in range(KG)]
            for d in gd:
                d.wait()
            sd = [pltpu.async_copy(
                      rows.at[j],
                      acc_sh.at[dst_v.at[pl.ds(c0 + j * CHUNK, CHUNK)]],
                      sem_s, add=True)
                  for j in range(KG)]
            for d in sd:
                d.wait()
            return 0
    else:
        def _grp(g, _):
            c0 = g * KG * CHUNK
            sd = [pltpu.async_copy(
                      rows.at[0],
                      acc_sh.at[dst_v.at[pl.ds(c0 + j * CHUNK, CHUNK)]],
                      sem_s, add=True)
                  for j in range(KG)]
            for d in sd:
                d.wait()
            return 0
    lax.fori_loop(0, NGRP, _grp, 0)
    plsc.subcore_barrier()

    for r in range(RPT // ZCH):
        row = sid * RPT + r * ZCH
        pltpu.sync_copy(acc_sh.at[pl.ds(row, ZCH)],
                        out_hbm.at[cid, pl.ds(row, ZCH)])


def _make_sc_pass(with_gather):
    return pl.kernel(
        functools.partial(_sc_pass_body, with_gather),
        out_type=jax.ShapeDtypeStruct((NC, NPAD, DH), jnp.float32),
        mesh=_mesh,
        scratch_types=[
            pltpu.VMEM((EPT,), jnp.int32),
            pltpu.VMEM((EPT,), jnp.int32),
            pltpu.VMEM((KG, CHUNK, DH), jnp.float32),
            pltpu.VMEM((ZCH, DH), jnp.float32),
            pltpu.VMEM_SHARED((NPAD, DH), jnp.float32),
            pltpu.VMEM_SHARED((NPAD, DH), jnp.float32),
            pltpu.SemaphoreType.DMA,
            pltpu.SemaphoreType.DMA,
        ],
        compiler_params=pltpu.CompilerParams(use_tc_tiling_on_sc=False),
    )


_sc_edge_pass = _make_sc_pass(True)
_sc_deg_pass = _make_sc_pass(False)


NP8 = NPAD // 8
NR8 = NN // 8


def _tc_mm0_body(x8_ref, w0b_ref, hw_ref):
    hw = jnp.dot(x8_ref[...], w0b_ref[...], preferred_element_type=jnp.float32)
    hw_ref[...] = jnp.concatenate([hw, jnp.zeros((NP8 - NR8, 128), jnp.float32)])


def _tc_scale_body(degp_ref, hw_ref, g_ref, dinv_ref):
    deg = degp_ref[0] + degp_ref[1] + 1.0
    dinv = lax.rsqrt(deg)
    dinv_ref[...] = dinv
    g_ref[...] = dinv * hw_ref[...]


def _tc_layer_body(accp_ref, hwp_ref, dinv_ref, b_ref, w_ref, hw_ref, g_ref):
    dinv = dinv_ref[...]
    h = dinv * (accp_ref[0] + accp_ref[1]) + dinv * dinv * hwp_ref[...] + b_ref[...]
    h = jnp.maximum(h, 0.0)
    hw = jnp.dot(h, w_ref[...], preferred_element_type=jnp.float32)
    hw_ref[...] = hw
    g_ref[...] = dinv * hw


def _tc_final_body(accp_ref, hwp_ref, dinv_ref, b_ref, out_ref):
    dinv = dinv_ref[...]
    out_ref[...] = (dinv * (accp_ref[0] + accp_ref[1])
                    + dinv * dinv * hwp_ref[...] + b_ref[...])


def kernel(x, edge_index, W0, b0, W1, b1, W2, b2):
    ei = edge_index.astype(jnp.int32)

    f32 = jnp.float32
    x8 = x.reshape(NR8, 8 * DIN)
    eye8 = jnp.eye(8, dtype=f32)
    W0b = jnp.kron(eye8, W0)
    Wb = [jnp.kron(eye8, W1), jnp.kron(eye8, W2), None]
    bb = [jnp.tile(b0, 8).reshape(1, 128), jnp.tile(b1, 8).reshape(1, 128),
          jnp.tile(b2, 8).reshape(1, 128)]
    g_dummy = jnp.zeros((NPAD, DH), f32)

    phys = jax.ShapeDtypeStruct((NP8, 128), f32)

    degp = _sc_deg_pass(g_dummy, ei).reshape(NC, NP8, 128)

    hw = pl.pallas_call(_tc_mm0_body, out_shape=[phys])(x8, W0b)[0]
    g, dinv = pl.pallas_call(
        _tc_scale_body,
        out_shape=[phys, phys],
    )(degp, hw)

    for i, last in ((0, False), (1, False), (2, True)):
        accp = _sc_edge_pass(g.reshape(NPAD, DH), ei).reshape(NC, NP8, 128)
        if last:
            out = pl.pallas_call(
                _tc_final_body,
                out_shape=[phys],
            )(accp, hw, dinv, bb[i])[0]
        else:
            hw, g = pl.pallas_call(
                _tc_layer_body,
                out_shape=[phys, phys],
            )(accp, hw, dinv, bb[i], Wb[i])

    return out.reshape(NPAD, DH)[:NN]


# speedup vs baseline: 88.2626x; 1.1390x over previous
import functools

import jax
import jax.numpy as jnp
from jax import lax
from jax.experimental import pallas as pl
from jax.experimental.pallas import tpu as pltpu
from jax.experimental.pallas import tpu_sc as plsc

NN = 10000
NPAD = 10240
NE = 320000
DH = 16
DIN = 128

NC, NS = 2, 16
NW = NC * NS
CHUNK = 80
NCHUNK = 125
EPT = NCHUNK * CHUNK
ZCH = 128
RPT = NPAD // NS
KG = 5
NGRP = NCHUNK // KG

_mesh = plsc.VectorSubcoreMesh(core_axis_name="c", subcore_axis_name="s")


def _sc_pass_body(with_gather, g_hbm, ei_hbm, out_hbm,
                  src_v, dst_v, rows, zbuf, acc_sh, g_sh, sem_g, sem_s):
    cid = lax.axis_index("c")
    sid = lax.axis_index("s")
    wid = sid * NC + cid

    def _zrow(i, _):
        zbuf[i] = jnp.zeros((DH,), jnp.float32)
        return 0
    lax.fori_loop(0, ZCH, _zrow, 0)
    for r in range(RPT // ZCH):
        pltpu.sync_copy(zbuf, acc_sh.at[pl.ds(sid * RPT + r * ZCH, ZCH)])

    base = wid * EPT
    pltpu.sync_copy(ei_hbm.at[1, pl.ds(base, EPT)], dst_v)
    if with_gather:
        pltpu.sync_copy(ei_hbm.at[0, pl.ds(base, EPT)], src_v)
        pltpu.sync_copy(g_hbm.at[pl.ds(sid * RPT, RPT)],
                        g_sh.at[pl.ds(sid * RPT, RPT)])
    else:
        def _orow(i, _):
            rows[0, i] = jnp.ones((DH,), jnp.float32)
            return 0
        lax.fori_loop(0, CHUNK, _orow, 0)
    plsc.subcore_barrier()

    def _wait_g():
        pltpu.make_async_copy(g_hbm.at[pl.ds(0, CHUNK)], rows.at[0], sem_g).wait()

    def _wait_s():
        pltpu.make_async_copy(g_hbm.at[pl.ds(0, CHUNK)], rows.at[0], sem_s).wait()

    def _fire_gathers(g, h):
        c0 = g * KG * CHUNK
        for j in range(KG):
            pltpu.async_copy(
                g_sh.at[src_v.at[pl.ds(c0 + j * CHUNK, CHUNK)]],
                rows.at[h * KG + j], sem_g)

    def _fire_scatters(g, h):
        c0 = g * KG * CHUNK
        for j in range(KG):
            pltpu.async_copy(
                rows.at[h * KG + j],
                acc_sh.at[dst_v.at[pl.ds(c0 + j * CHUNK, CHUNK)]],
                sem_s, add=True)

    if with_gather:
        _fire_gathers(0, 0)

        def _grp(g, _):
            h = g % 2

            @pl.when(g >= 1)
            def _():
                for _j in range(KG):
                    _wait_s()

            @pl.when(g < NGRP - 1)
            def _():
                _fire_gathers(g + 1, 1 - h)
            for _j in range(KG):
                _wait_g()
            _fire_scatters(g, h)
            return 0
        lax.fori_loop(0, NGRP, _grp, 0)
        for _j in range(KG):
            _wait_s()
    else:
        def _grp(g, _):
            c0 = g * KG * CHUNK

            @pl.when(g >= 1)
            def _():
                for _j in range(KG):
                    _wait_s()
            for j in range(KG):
                pltpu.async_copy(
                    rows.at[0],
                    acc_sh.at[dst_v.at[pl.ds(c0 + j * CHUNK, CHUNK)]],
                    sem_s, add=True)
            return 0
        lax.fori_loop(0, NGRP, _grp, 0)
        for _j in range(KG):
            _wait_s()
    plsc.subcore_barrier()

    for r in range(RPT // ZCH):
        row = sid * RPT + r * ZCH
        pltpu.sync_copy(acc_sh.at[pl.ds(row, ZCH)],
                        out_hbm.at[cid, pl.ds(row, ZCH)])


def _make_sc_pass(with_gather):
    return pl.kernel(
        functools.partial(_sc_pass_body, with_gather),
        out_type=jax.ShapeDtypeStruct((NC, NPAD, DH), jnp.float32),
        mesh=_mesh,
        scratch_types=[
            pltpu.VMEM((EPT,), jnp.int32),
            pltpu.VMEM((EPT,), jnp.int32),
            pltpu.VMEM((2 * KG, CHUNK, DH), jnp.float32),
            pltpu.VMEM((ZCH, DH), jnp.float32),
            pltpu.VMEM_SHARED((NPAD, DH), jnp.float32),
            pltpu.VMEM_SHARED((NPAD, DH), jnp.float32),
            pltpu.SemaphoreType.DMA,
            pltpu.SemaphoreType.DMA,
        ],
        compiler_params=pltpu.CompilerParams(use_tc_tiling_on_sc=False),
    )


_sc_edge_pass = _make_sc_pass(True)
_sc_deg_pass = _make_sc_pass(False)


NP8 = NPAD // 8
NR8 = NN // 8


def _tc_mm0_body(x8_ref, w0b_ref, hw_ref):
    hw = jnp.dot(x8_ref[...], w0b_ref[...], preferred_element_type=jnp.float32)
    hw_ref[...] = jnp.concatenate([hw, jnp.zeros((NP8 - NR8, 128), jnp.float32)])


def _tc_scale_body(degp_ref, hw_ref, g_ref, dinv_ref):
    deg = degp_ref[0] + degp_ref[1] + 1.0
    dinv = lax.rsqrt(deg)
    dinv_ref[...] = dinv
    g_ref[...] = dinv * hw_ref[...]


def _tc_layer_body(accp_ref, hwp_ref, dinv_ref, b_ref, w_ref, hw_ref, g_ref):
    dinv = dinv_ref[...]
    h = dinv * (accp_ref[0] + accp_ref[1]) + dinv * dinv * hwp_ref[...] + b_ref[...]
    h = jnp.maximum(h, 0.0)
    hw = jnp.dot(h, w_ref[...], preferred_element_type=jnp.float32)
    hw_ref[...] = hw
    g_ref[...] = dinv * hw


def _tc_final_body(accp_ref, hwp_ref, dinv_ref, b_ref, out_ref):
    dinv = dinv_ref[...]
    out_ref[...] = (dinv * (accp_ref[0] + accp_ref[1])
                    + dinv * dinv * hwp_ref[...] + b_ref[...])


def kernel(x, edge_index, W0, b0, W1, b1, W2, b2):
    ei = edge_index.astype(jnp.int32)

    f32 = jnp.float32
    x8 = x.reshape(NR8, 8 * DIN)
    eye8 = jnp.eye(8, dtype=f32)
    W0b = jnp.kron(eye8, W0)
    Wb = [jnp.kron(eye8, W1), jnp.kron(eye8, W2), None]
    bb = [jnp.tile(b0, 8).reshape(1, 128), jnp.tile(b1, 8).reshape(1, 128),
          jnp.tile(b2, 8).reshape(1, 128)]
    g_dummy = jnp.zeros((NPAD, DH), f32)

    phys = jax.ShapeDtypeStruct((NP8, 128), f32)

    degp = _sc_deg_pass(g_dummy, ei).reshape(NC, NP8, 128)

    hw = pl.pallas_call(_tc_mm0_body, out_shape=[phys])(x8, W0b)[0]
    g, dinv = pl.pallas_call(
        _tc_scale_body,
        out_shape=[phys, phys],
    )(degp, hw)

    for i, last in ((0, False), (1, False), (2, True)):
        accp = _sc_edge_pass(g.reshape(NPAD, DH), ei).reshape(NC, NP8, 128)
        if last:
            out = pl.pallas_call(
                _tc_final_body,
                out_shape=[phys],
            )(accp, hw, dinv, bb[i])[0]
        else:
            hw, g = pl.pallas_call(
                _tc_layer_body,
                out_shape=[phys, phys],
            )(accp, hw, dinv, bb[i], Wb[i])

    return out.reshape(NPAD, DH)[:NN]
